# baseline (speedup 1.0000x reference)
import jax
import jax.numpy as jnp
from jax.experimental import pallas as pl
from jax.experimental.pallas import tpu as pltpu

_BM = 200


def _fused_kernel(x_ref, a0_ref, a1_ref, wt_ref, b_ref, out_ref, y_ref):
    i = pl.program_id(0)

    @pl.when(i == 0)
    def _():
        xb = x_ref[...].astype(jnp.bfloat16)
        wb = wt_ref[...].astype(jnp.bfloat16)
        y_ref[...] = jnp.dot(
            xb, wb, preferred_element_type=jnp.float32
        ).astype(jnp.bfloat16)

    @pl.when(i > 0)
    def _():
        bias = b_ref[...]
        y = y_ref[...]
        out_ref[0] = (
            jnp.dot(
                a0_ref[0].astype(jnp.bfloat16),
                y,
                preferred_element_type=jnp.float32,
            )
            + bias
        )
        out_ref[1] = (
            jnp.dot(
                a1_ref[0].astype(jnp.bfloat16),
                y,
                preferred_element_type=jnp.float32,
            )
            + bias
        )


def kernel(x, adj, W, b):
    n, in_f = x.shape
    out_f = W.shape[0]
    wt = W.T
    b2 = b.reshape(1, out_f)
    adj3 = adj.reshape(2, n // 2, n)

    def _row(i):
        return jnp.maximum(i - 1, 0)

    out = pl.pallas_call(
        _fused_kernel,
        grid=(1 + (n // 2) // _BM,),
        in_specs=[
            pl.BlockSpec((n, in_f), lambda i: (0, 0)),
            pl.BlockSpec((1, _BM, n), lambda i: (0, _row(i), 0)),
            pl.BlockSpec((1, _BM, n), lambda i: (1, _row(i), 0)),
            pl.BlockSpec((in_f, out_f), lambda i: (0, 0)),
            pl.BlockSpec((1, out_f), lambda i: (0, 0)),
        ],
        out_specs=pl.BlockSpec((2, _BM, out_f), lambda i: (0, _row(i), 0)),
        out_shape=jax.ShapeDtypeStruct((2, n // 2, out_f), jnp.float32),
        scratch_shapes=[pltpu.VMEM((n, out_f), jnp.bfloat16)],
        compiler_params=pltpu.CompilerParams(
            dimension_semantics=("arbitrary",),
            vmem_limit_bytes=62 * 1024 * 1024,
        ),
    )(x, adj3, adj3, wt, b2)
    return out.reshape(n, out_f)
